# speedup vs baseline: 1.3351x; 1.3351x over previous
import functools

import jax
import jax.numpy as jnp
from jax import lax
from jax.experimental import pallas as pl
from jax.experimental.pallas import tpu as pltpu
from jax.experimental.pallas import tpu_sc as plsc

VOCAB = 50257
N_EMBD = 768
BATCH = 4
SEQ = 2048
TOKENS = BATCH * SEQ
NUM_CORES = 2
NUM_SUBCORES = 16
NW = NUM_CORES * NUM_SUBCORES
TOK_PER_W = TOKENS // NW
CHUNK = 64
N_CHUNKS = TOK_PER_W // CHUNK
LANES = 16
SLICES = N_EMBD // LANES


def _sc_body(ids_hbm, wte_hbm, wpe_hbm, out_hbm, idx_v, wte_v, wpe_v, sem):
    wid = lax.axis_index("s") * NUM_CORES + lax.axis_index("c")
    base = wid * TOK_PER_W
    seq0 = lax.rem(base, SEQ)

    for c in range(N_CHUNKS):
        tok0 = base + c * CHUNK
        pltpu.sync_copy(ids_hbm.at[pl.ds(tok0, CHUNK)], idx_v)
        gather = pltpu.async_copy(wte_hbm.at[idx_v], wte_v, sem)
        pltpu.sync_copy(wpe_hbm.at[pl.ds(seq0 + c * CHUNK, CHUNK)], wpe_v)
        gather.wait()

        def row_add(r, _):
            for j in range(SLICES):
                sl = pl.ds(j * LANES, LANES)
                wpe_v[r, sl] = wpe_v[r, sl] + wte_v[r, sl]
            return 0

        lax.fori_loop(0, CHUNK, row_add, 0)
        pltpu.sync_copy(wpe_v, out_hbm.at[pl.ds(tok0, CHUNK)])


_sc_kernel = functools.partial(
    pl.kernel,
    mesh=plsc.VectorSubcoreMesh(core_axis_name="c", subcore_axis_name="s"),
    out_type=jax.ShapeDtypeStruct((TOKENS, N_EMBD), jnp.float32),
    scratch_types=[
        pltpu.VMEM((CHUNK,), jnp.int32),
        pltpu.VMEM((CHUNK, N_EMBD), jnp.float32),
        pltpu.VMEM((CHUNK, N_EMBD), jnp.float32),
        pltpu.SemaphoreType.DMA,
    ],
)(_sc_body)


def kernel(input_ids, wte, wpe):
    ids_flat = input_ids.reshape(TOKENS)
    out = _sc_kernel(ids_flat, wte, wpe)
    return out.reshape(BATCH, SEQ, N_EMBD)
